# Initial kernel scaffold; baseline (speedup 1.0000x reference)
import functools

import jax
import jax.numpy as jnp
from jax import lax
from jax.experimental import pallas as pl
from jax.experimental.pallas import tpu as pltpu
from jax.experimental.pallas import tpu_sc as plsc

_NC = 2
_NS = 16
_NW = _NC * _NS
_C = 128
_K = 10


@functools.lru_cache(maxsize=None)
def _make_gather(B, V, D):
    assert B % (_NW * _C) == 0
    bpw = B // _NW
    steps = bpw // _C
    assert steps % _K == 0
    nblk = steps // _K
    cb = _C * _K

    mesh = plsc.VectorSubcoreMesh(core_axis_name="c", subcore_axis_name="s")

    @functools.partial(
        pl.kernel,
        out_type=jax.ShapeDtypeStruct((B, D), jnp.float32),
        mesh=mesh,
        scratch_types=[
            pltpu.VMEM((steps, _C), jnp.int32),
            pltpu.VMEM((cb, D), jnp.float32),
            pltpu.SemaphoreType.DMA,
        ],
    )
    def gather_kernel(x_hbm, table_hbm, out_hbm, idx_v, rows_v, sem_g):
        wid = lax.axis_index("s") * _NC + lax.axis_index("c")
        base = wid * bpw
        pltpu.sync_copy(x_hbm.at[wid], idx_v)

        def blk(i, carry):
            for j in range(_K):
                pltpu.make_async_copy(
                    table_hbm.at[idx_v.at[i * _K + j]],
                    rows_v.at[pl.ds(j * _C, _C)],
                    sem_g,
                ).start()
            for j in range(_K):
                pltpu.make_async_copy(
                    table_hbm.at[idx_v.at[i * _K + j]],
                    rows_v.at[pl.ds(j * _C, _C)],
                    sem_g,
                ).wait()
            pltpu.sync_copy(rows_v, out_hbm.at[pl.ds(base + i * cb, cb)])
            return carry

        lax.fori_loop(0, nblk, blk, 0)

    return gather_kernel


def kernel(x, table):
    b, h = x.shape
    v, d = table.shape
    flat = b * h
    xf = x.reshape(_NW, flat // (_NW * _C), _C).astype(jnp.int32)
    out = _make_gather(flat, v, d)(xf, table)
    return out.reshape(b, h, d)


# speedup vs baseline: 1.1046x; 1.1046x over previous
import functools

import jax
import jax.numpy as jnp
from jax import lax
from jax.experimental import pallas as pl
from jax.experimental.pallas import tpu as pltpu
from jax.experimental.pallas import tpu_sc as plsc

_NC = 2
_NS = 16
_NW = _NC * _NS
_C = 128
_K = 10


@functools.lru_cache(maxsize=None)
def _make_gather(B, V, D):
    assert B % (_NW * _C) == 0
    bpw = B // _NW
    steps = bpw // _C
    assert steps % _K == 0
    nblk = steps // _K
    cb = _C * _K

    mesh = plsc.VectorSubcoreMesh(core_axis_name="c", subcore_axis_name="s")

    @functools.partial(
        pl.kernel,
        out_type=jax.ShapeDtypeStruct((B, D), jnp.float32),
        mesh=mesh,
        scratch_types=[
            pltpu.VMEM((steps, _C), jnp.int32),
            pltpu.VMEM((cb, D), jnp.float32),
            pltpu.SemaphoreType.DMA,
        ],
        compiler_params=pltpu.CompilerParams(use_tc_tiling_on_sc=False),
    )
    def gather_kernel(x_hbm, table_hbm, out_hbm, idx_v, rows_v, sem_g):
        wid = lax.axis_index("s") * _NC + lax.axis_index("c")
        base = wid * bpw
        pltpu.sync_copy(x_hbm.at[wid], idx_v)

        def blk(i, carry):
            for j in range(_K):
                pltpu.make_async_copy(
                    table_hbm.at[idx_v.at[i * _K + j]],
                    rows_v.at[pl.ds(j * _C, _C)],
                    sem_g,
                ).start()
            for j in range(_K):
                pltpu.make_async_copy(
                    table_hbm.at[idx_v.at[i * _K + j]],
                    rows_v.at[pl.ds(j * _C, _C)],
                    sem_g,
                ).wait()
            pltpu.sync_copy(rows_v, out_hbm.at[pl.ds(base + i * cb, cb)])
            return carry

        lax.fori_loop(0, nblk, blk, 0)

    return gather_kernel


def kernel(x, table):
    b, h = x.shape
    v, d = table.shape
    flat = b * h
    xf = x.reshape(_NW, flat // (_NW * _C), _C).astype(jnp.int32)
    out = _make_gather(flat, v, d)(xf, table)
    return out.reshape(b, h, d)


# speedup vs baseline: 1.7711x; 1.6033x over previous
import functools

import jax
import jax.numpy as jnp
from jax import lax
from jax.experimental import pallas as pl
from jax.experimental.pallas import tpu as pltpu
from jax.experimental.pallas import tpu_sc as plsc

_NC = 2
_NS = 16
_NW = _NC * _NS
_RB = 16


@functools.lru_cache(maxsize=None)
def _make_gather(B, H, V, D):
    assert B % _NW == 0
    bpw = B // _NW
    assert bpw % _RB == 0
    nblk = bpw // _RB

    mesh = plsc.VectorSubcoreMesh(core_axis_name="c", subcore_axis_name="s")

    @functools.partial(
        pl.kernel,
        out_type=jax.ShapeDtypeStruct((B, H, D), jnp.float32),
        mesh=mesh,
        scratch_types=[
            pltpu.VMEM((bpw, H), jnp.int32),
            pltpu.VMEM((_RB, H, D), jnp.float32),
            pltpu.SemaphoreType.DMA,
        ],
        compiler_params=pltpu.CompilerParams(use_tc_tiling_on_sc=False),
    )
    def gather_kernel(x_hbm, table_hbm, out_hbm, idx_v, rows_v, sem_g):
        wid = lax.axis_index("s") * _NC + lax.axis_index("c")
        xbase = wid * bpw
        pltpu.sync_copy(x_hbm.at[pl.ds(xbase, bpw)], idx_v)

        def blk(i, carry):
            for j in range(_RB):
                pltpu.make_async_copy(
                    table_hbm.at[idx_v.at[i * _RB + j]], rows_v.at[j], sem_g
                ).start()
            for j in range(_RB):
                pltpu.make_async_copy(
                    table_hbm.at[idx_v.at[i * _RB + j]], rows_v.at[j], sem_g
                ).wait()
            pltpu.sync_copy(rows_v, out_hbm.at[pl.ds(xbase + i * _RB, _RB)])
            return carry

        lax.fori_loop(0, nblk, blk, 0)

    return gather_kernel


def kernel(x, table):
    b, h = x.shape
    v, d = table.shape
    return _make_gather(b, h, v, d)(x, table)
